# baseline (speedup 1.0000x reference)
import functools

import jax
import jax.numpy as jnp
from jax import lax
from jax.experimental import pallas as pl
from jax.experimental.pallas import tpu as pltpu
from jax.experimental.pallas import tpu_sc as plsc

ROWS = 128
COLS = 100000
NC = 2
NS = 16
NW = NC * NS
ROWS_PER_W = ROWS // NW
CHUNK = 4000
NCHUNKS = COLS // CHUNK
VECS = CHUNK // 16

EPS = 1e-11
SHIFT = 16.0
SQRT_BITS = 0x3F3504F3
NEG_LN2 = -0.6931471805599453
P3 = -0.2281064108
P2 = 0.3545640653
P1 = -0.5019703550
P0 = 0.9997000880


def _neg_log(v):
    bits = lax.bitcast_convert_type(v, jnp.int32)
    bp = bits - SQRT_BITS
    e = lax.shift_right_arithmetic(bp, 23)
    fb = (bp & 0x007FFFFF) + SQRT_BITS
    f = lax.bitcast_convert_type(fb, jnp.float32)
    ef = e.astype(jnp.float32)
    r = f - 1.0
    p = ((P3 * r + P2) * r + P1) * r + P0
    return ef * NEG_LN2 - r * p


def _sc_body(x_hbm, u_hbm, o_hbm, nbuf, xb0, xb1, ub0, ub1, sems):
    wid = lax.axis_index("s") * NC + lax.axis_index("c")

    def start_in(ci, row, xb, ub, slot):
        c0 = pl.multiple_of(row * COLS + ci * CHUNK, 8)
        pltpu.make_async_copy(x_hbm.at[pl.ds(c0, CHUNK)], xb,
                              sems.at[slot]).start()
        pltpu.make_async_copy(u_hbm.at[pl.ds(c0, CHUNK)], ub,
                              sems.at[slot + 1]).start()

    def wait_in(ci, row, xb, ub, slot):
        c0 = pl.multiple_of(row * COLS + ci * CHUNK, 8)
        pltpu.make_async_copy(x_hbm.at[pl.ds(c0, CHUNK)], xb,
                              sems.at[slot]).wait()
        pltpu.make_async_copy(u_hbm.at[pl.ds(c0, CHUNK)], ub,
                              sems.at[slot + 1]).wait()

    def chunk_compute(ci, xb, ub, acc):
        base = ci * CHUNK

        def body(v, acc):
            sl = pl.ds(v * 16, 16)
            x16 = xb[sl]
            u16 = ub[sl]
            t = _neg_log(u16 + EPS)
            n = jnp.exp(x16 - SHIFT) / t
            nbuf[pl.ds(base + v * 16, 16)] = n
            return acc + n

        return lax.fori_loop(0, VECS, body, acc, unroll=4)

    for ri in range(ROWS_PER_W):
        row = wid * ROWS_PER_W + ri
        start_in(0, row, xb0, ub0, 0)

        def pair(k, acc):
            start_in(2 * k + 1, row, xb1, ub1, 2)
            wait_in(2 * k, row, xb0, ub0, 0)
            acc = chunk_compute(2 * k, xb0, ub0, acc)
            start_in(2 * k + 2, row, xb0, ub0, 0)
            wait_in(2 * k + 1, row, xb1, ub1, 2)
            return chunk_compute(2 * k + 1, xb1, ub1, acc)

        acc = lax.fori_loop(0, (NCHUNKS - 1) // 2, pair,
                            jnp.zeros((16,), jnp.float32))
        wait_in(NCHUNKS - 1, row, xb0, ub0, 0)
        acc = chunk_compute(NCHUNKS - 1, xb0, ub0, acc)

        for _k in (1, 2, 4, 8):
            _idx = jax.lax.iota(jnp.int32, 16) ^ _k
            _perm = jax.lax.gather(
                acc, _idx[:, None],
                jax.lax.GatherDimensionNumbers(
                    offset_dims=(), collapsed_slice_dims=(0,),
                    start_index_map=(0,)),
                (1,), mode=jax.lax.GatherScatterMode.PROMISE_IN_BOUNDS)
            acc = acc + _perm
        inv = 1.0 / acc

        def scale(v, _):
            sl = pl.ds(v * 16, 16)
            nbuf[sl] = nbuf[sl] * inv
            return 0

        lax.fori_loop(0, COLS // 16, scale, 0, unroll=8)
        ob = pl.multiple_of(row * COLS, 8)
        pltpu.sync_copy(nbuf, o_hbm.at[pl.ds(ob, COLS)])


@functools.partial(jax.jit)
def kernel(logits, u):
    mesh = plsc.VectorSubcoreMesh(core_axis_name="c", subcore_axis_name="s")
    f = functools.partial(
        pl.kernel,
        out_type=jax.ShapeDtypeStruct((ROWS * COLS,), jnp.float32),
        mesh=mesh,
        scratch_types=[
            pltpu.VMEM((COLS,), jnp.float32),
            pltpu.VMEM((CHUNK,), jnp.float32),
            pltpu.VMEM((CHUNK,), jnp.float32),
            pltpu.VMEM((CHUNK,), jnp.float32),
            pltpu.VMEM((CHUNK,), jnp.float32),
            pltpu.SemaphoreType.DMA((4,)),
        ],
    )(_sc_body)
    out = f(logits.reshape(ROWS * COLS), u.reshape(ROWS * COLS))
    return out.reshape(ROWS, COLS)


# speedup vs baseline: 1.0063x; 1.0063x over previous
import functools

import jax
import jax.numpy as jnp
from jax import lax
from jax.experimental import pallas as pl
from jax.experimental.pallas import tpu as pltpu
from jax.experimental.pallas import tpu_sc as plsc

ROWS = 128
COLS = 100000
NC = 2
NS = 16
NW = NC * NS
ROWS_PER_W = ROWS // NW
CHUNK = 4000
NCHUNKS = COLS // CHUNK
VECS = CHUNK // 16

EPS = 1e-11
SHIFT = 16.0
SQRT_BITS = 0x3F3504F3
NEG_LN2 = -0.6931471805599453
P3 = -0.2281064108
P2 = 0.3545640653
P1 = -0.5019703550
P0 = 0.9997000880


def _neg_log(v):
    bits = lax.bitcast_convert_type(v, jnp.int32)
    bp = bits - SQRT_BITS
    e = lax.shift_right_arithmetic(bp, 23)
    fb = (bp & 0x007FFFFF) + SQRT_BITS
    f = lax.bitcast_convert_type(fb, jnp.float32)
    ef = e.astype(jnp.float32)
    r = f - 1.0
    p = ((P3 * r + P2) * r + P1) * r + P0
    return ef * NEG_LN2 - r * p


def _sc_body(x_hbm, u_hbm, o_hbm, nbuf, xb0, xb1, ub0, ub1, sems):
    wid = lax.axis_index("s") * NC + lax.axis_index("c")

    def start_in(ci, row, xb, ub, slot):
        c0 = pl.multiple_of(row * COLS + ci * CHUNK, 8)
        pltpu.make_async_copy(x_hbm.at[pl.ds(c0, CHUNK)], xb,
                              sems.at[slot]).start()
        pltpu.make_async_copy(u_hbm.at[pl.ds(c0, CHUNK)], ub,
                              sems.at[slot + 1]).start()

    def wait_in(ci, row, xb, ub, slot):
        c0 = pl.multiple_of(row * COLS + ci * CHUNK, 8)
        pltpu.make_async_copy(x_hbm.at[pl.ds(c0, CHUNK)], xb,
                              sems.at[slot]).wait()
        pltpu.make_async_copy(u_hbm.at[pl.ds(c0, CHUNK)], ub,
                              sems.at[slot + 1]).wait()

    def chunk_compute(ci, xb, ub, acc):
        base = ci * CHUNK

        def body(i, acc):
            sl = pl.ds(i, 16)
            x16 = xb[sl]
            u16 = ub[sl]
            t = _neg_log(u16 + EPS)
            n = jnp.exp(x16 - SHIFT) / t
            nbuf[pl.ds(base + i, 16)] = n
            return acc + n

        return plsc.parallel_loop(0, CHUNK, step=16, unroll=8,
                                  carry=acc)(body)

    for ri in range(ROWS_PER_W):
        row = wid * ROWS_PER_W + ri
        start_in(0, row, xb0, ub0, 0)

        def pair(k, acc):
            start_in(2 * k + 1, row, xb1, ub1, 2)
            wait_in(2 * k, row, xb0, ub0, 0)
            acc = chunk_compute(2 * k, xb0, ub0, acc)
            start_in(2 * k + 2, row, xb0, ub0, 0)
            wait_in(2 * k + 1, row, xb1, ub1, 2)
            return chunk_compute(2 * k + 1, xb1, ub1, acc)

        acc = lax.fori_loop(0, (NCHUNKS - 1) // 2, pair,
                            jnp.zeros((16,), jnp.float32))
        wait_in(NCHUNKS - 1, row, xb0, ub0, 0)
        acc = chunk_compute(NCHUNKS - 1, xb0, ub0, acc)

        for _k in (1, 2, 4, 8):
            _idx = jax.lax.iota(jnp.int32, 16) ^ _k
            _perm = jax.lax.gather(
                acc, _idx[:, None],
                jax.lax.GatherDimensionNumbers(
                    offset_dims=(), collapsed_slice_dims=(0,),
                    start_index_map=(0,)),
                (1,), mode=jax.lax.GatherScatterMode.PROMISE_IN_BOUNDS)
            acc = acc + _perm
        inv = 1.0 / acc

        @plsc.parallel_loop(0, COLS, step=16, unroll=8)
        def scale(i):
            sl = pl.ds(i, 16)
            nbuf[sl] = nbuf[sl] * inv
        ob = pl.multiple_of(row * COLS, 8)
        pltpu.sync_copy(nbuf, o_hbm.at[pl.ds(ob, COLS)])


@functools.partial(jax.jit)
def kernel(logits, u):
    mesh = plsc.VectorSubcoreMesh(core_axis_name="c", subcore_axis_name="s")
    f = functools.partial(
        pl.kernel,
        out_type=jax.ShapeDtypeStruct((ROWS * COLS,), jnp.float32),
        mesh=mesh,
        scratch_types=[
            pltpu.VMEM((COLS,), jnp.float32),
            pltpu.VMEM((CHUNK,), jnp.float32),
            pltpu.VMEM((CHUNK,), jnp.float32),
            pltpu.VMEM((CHUNK,), jnp.float32),
            pltpu.VMEM((CHUNK,), jnp.float32),
            pltpu.SemaphoreType.DMA((4,)),
        ],
    )(_sc_body)
    out = f(logits.reshape(ROWS * COLS), u.reshape(ROWS * COLS))
    return out.reshape(ROWS, COLS)


# speedup vs baseline: 1.0113x; 1.0050x over previous
import functools

import jax
import jax.numpy as jnp
from jax import lax
from jax.experimental import pallas as pl
from jax.experimental.pallas import tpu as pltpu
from jax.experimental.pallas import tpu_sc as plsc

ROWS = 128
COLS = 100000
NC = 2
NS = 16
NW = NC * NS
ROWS_PER_W = ROWS // NW
CHUNK = 4000
NCHUNKS = COLS // CHUNK
VECS = CHUNK // 16
UNROLL = 4

EPS = 1e-11
SHIFT = 16.0
SQRT_BITS = 0x3F3504F3
NEG_LN2 = -0.6931471805599453
P3 = -0.2281064108
P2 = 0.3545640653
P1 = -0.5019703550
P0 = 0.9997000880


def _neg_log(v):
    bits = lax.bitcast_convert_type(v, jnp.int32)
    bp = bits - SQRT_BITS
    e = lax.shift_right_arithmetic(bp, 23)
    fb = (bp & 0x007FFFFF) + SQRT_BITS
    f = lax.bitcast_convert_type(fb, jnp.float32)
    ef = e.astype(jnp.float32)
    r = f - 1.0
    p = ((P3 * r + P2) * r + P1) * r + P0
    return ef * NEG_LN2 - r * p


def _sc_body(x_hbm, u_hbm, o_hbm, nbuf, xb0, xb1, ub0, ub1, sems):
    wid = lax.axis_index("s") * NC + lax.axis_index("c")

    def start_in(ci, row, xb, ub, slot):
        c0 = pl.multiple_of(row * COLS + ci * CHUNK, 8)
        pltpu.make_async_copy(x_hbm.at[pl.ds(c0, CHUNK)], xb,
                              sems.at[slot]).start()
        pltpu.make_async_copy(u_hbm.at[pl.ds(c0, CHUNK)], ub,
                              sems.at[slot + 1]).start()

    def wait_in(ci, row, xb, ub, slot):
        c0 = pl.multiple_of(row * COLS + ci * CHUNK, 8)
        pltpu.make_async_copy(x_hbm.at[pl.ds(c0, CHUNK)], xb,
                              sems.at[slot]).wait()
        pltpu.make_async_copy(u_hbm.at[pl.ds(c0, CHUNK)], ub,
                              sems.at[slot + 1]).wait()

    def chunk_compute(ci, xb, ub, acc):
        base = ci * CHUNK

        def body(i, acc):
            ns = []
            for j in range(UNROLL):
                sl = pl.ds(i + j * 16, 16)
                t = _neg_log(ub[sl])
                n = jnp.exp(xb[sl] - SHIFT) / t
                nbuf[pl.ds(base + i + j * 16, 16)] = n
                ns.append(n)
            return acc + ((ns[0] + ns[1]) + (ns[2] + ns[3]))

        return plsc.parallel_loop(0, CHUNK, step=16 * UNROLL,
                                  carry=acc)(body)

    for ri in range(ROWS_PER_W):
        row = wid * ROWS_PER_W + ri
        start_in(0, row, xb0, ub0, 0)

        def pair(k, acc):
            start_in(2 * k + 1, row, xb1, ub1, 2)
            wait_in(2 * k, row, xb0, ub0, 0)
            acc = chunk_compute(2 * k, xb0, ub0, acc)
            start_in(2 * k + 2, row, xb0, ub0, 0)
            wait_in(2 * k + 1, row, xb1, ub1, 2)
            return chunk_compute(2 * k + 1, xb1, ub1, acc)

        acc = lax.fori_loop(0, (NCHUNKS - 1) // 2, pair,
                            jnp.zeros((16,), jnp.float32))
        wait_in(NCHUNKS - 1, row, xb0, ub0, 0)
        acc = chunk_compute(NCHUNKS - 1, xb0, ub0, acc)

        for _k in (1, 2, 4, 8):
            _idx = jax.lax.iota(jnp.int32, 16) ^ _k
            _perm = jax.lax.gather(
                acc, _idx[:, None],
                jax.lax.GatherDimensionNumbers(
                    offset_dims=(), collapsed_slice_dims=(0,),
                    start_index_map=(0,)),
                (1,), mode=jax.lax.GatherScatterMode.PROMISE_IN_BOUNDS)
            acc = acc + _perm
        inv = 1.0 / acc

        @plsc.parallel_loop(0, COLS, step=16 * UNROLL)
        def scale(i):
            for j in range(UNROLL):
                sl = pl.ds(i + j * 16, 16)
                nbuf[sl] = nbuf[sl] * inv
        ob = pl.multiple_of(row * COLS, 8)
        pltpu.sync_copy(nbuf, o_hbm.at[pl.ds(ob, COLS)])


@functools.partial(jax.jit)
def kernel(logits, u):
    mesh = plsc.VectorSubcoreMesh(core_axis_name="c", subcore_axis_name="s")
    f = functools.partial(
        pl.kernel,
        out_type=jax.ShapeDtypeStruct((ROWS * COLS,), jnp.float32),
        mesh=mesh,
        scratch_types=[
            pltpu.VMEM((COLS,), jnp.float32),
            pltpu.VMEM((CHUNK,), jnp.float32),
            pltpu.VMEM((CHUNK,), jnp.float32),
            pltpu.VMEM((CHUNK,), jnp.float32),
            pltpu.VMEM((CHUNK,), jnp.float32),
            pltpu.SemaphoreType.DMA((4,)),
        ],
    )(_sc_body)
    out = f(logits.reshape(ROWS * COLS), u.reshape(ROWS * COLS))
    return out.reshape(ROWS, COLS)


# speedup vs baseline: 1.8525x; 1.8318x over previous
import functools

import jax
import jax.numpy as jnp
from jax import lax
from jax.experimental import pallas as pl
from jax.experimental.pallas import tpu as pltpu
from jax.experimental.pallas import tpu_sc as plsc

ROWS = 128
COLS = 100000
NC = 2
NS = 16
NW = NC * NS
ROWS_PER_W = ROWS // NW
CHUNK = 4000
NCHUNKS = COLS // CHUNK
VECS = CHUNK // 16
UNROLL = 4

EPS = 1e-11
SHIFT = 16.0
SQRT_BITS = 0x3F3504F3
NEG_LN2 = -0.6931471805599453
P3 = -0.2281064108
P2 = 0.3545640653
P1 = -0.5019703550
P0 = 0.9997000880


def _neg_log(v):
    bits = lax.bitcast_convert_type(v, jnp.int32)
    bp = bits - SQRT_BITS
    e = lax.shift_right_arithmetic(bp, 23)
    fb = (bp & 0x007FFFFF) + SQRT_BITS
    f = lax.bitcast_convert_type(fb, jnp.float32)
    ef = e.astype(jnp.float32)
    r = f - 1.0
    p = ((P3 * r + P2) * r + P1) * r + P0
    return ef * NEG_LN2 - r * p


def _sc_body(x_hbm, u_hbm, o_hbm, nbuf, xb0, xb1, ub0, ub1, sems):
    wid = lax.axis_index("s") * NC + lax.axis_index("c")

    def start_in(ci, row, xb, ub, slot):
        c0 = pl.multiple_of(row * COLS + ci * CHUNK, 8)
        pltpu.make_async_copy(x_hbm.at[pl.ds(c0, CHUNK)], xb,
                              sems.at[slot]).start()
        pltpu.make_async_copy(u_hbm.at[pl.ds(c0, CHUNK)], ub,
                              sems.at[slot + 1]).start()

    def wait_in(ci, row, xb, ub, slot):
        c0 = pl.multiple_of(row * COLS + ci * CHUNK, 8)
        pltpu.make_async_copy(x_hbm.at[pl.ds(c0, CHUNK)], xb,
                              sems.at[slot]).wait()
        pltpu.make_async_copy(u_hbm.at[pl.ds(c0, CHUNK)], ub,
                              sems.at[slot + 1]).wait()

    def chunk_compute(ci, xb, ub, accs):
        base = ci * CHUNK

        @plsc.parallel_loop(0, CHUNK, step=16, unroll=5)
        def pass1(i):
            t = _neg_log(ub[pl.ds(i, 16)])
            n = jnp.exp(xb[pl.ds(i, 16)] - SHIFT) / t
            nbuf[pl.ds(base + i, 16)] = n

        def body(v, accs):
            a0, a1 = accs
            i = base + v * 32
            return (a0 + nbuf[pl.ds(i, 16)],
                    a1 + nbuf[pl.ds(i + 16, 16)])

        return lax.fori_loop(0, CHUNK // 32, body, accs)

    for ri in range(ROWS_PER_W):
        row = wid * ROWS_PER_W + ri
        start_in(0, row, xb0, ub0, 0)

        def pair(k, accs):
            start_in(2 * k + 1, row, xb1, ub1, 2)
            wait_in(2 * k, row, xb0, ub0, 0)
            accs = chunk_compute(2 * k, xb0, ub0, accs)
            start_in(2 * k + 2, row, xb0, ub0, 0)
            wait_in(2 * k + 1, row, xb1, ub1, 2)
            return chunk_compute(2 * k + 1, xb1, ub1, accs)

        zero = jnp.zeros((16,), jnp.float32)
        accs = lax.fori_loop(0, (NCHUNKS - 1) // 2, pair, (zero, zero))
        wait_in(NCHUNKS - 1, row, xb0, ub0, 0)
        accs = chunk_compute(NCHUNKS - 1, xb0, ub0, accs)
        acc = accs[0] + accs[1]

        for _k in (1, 2, 4, 8):
            _idx = jax.lax.iota(jnp.int32, 16) ^ _k
            _perm = jax.lax.gather(
                acc, _idx[:, None],
                jax.lax.GatherDimensionNumbers(
                    offset_dims=(), collapsed_slice_dims=(0,),
                    start_index_map=(0,)),
                (1,), mode=jax.lax.GatherScatterMode.PROMISE_IN_BOUNDS)
            acc = acc + _perm
        inv = 1.0 / acc

        @plsc.parallel_loop(0, COLS, step=16, unroll=5)
        def scale(i):
            sl = pl.ds(i, 16)
            nbuf[sl] = nbuf[sl] * inv
        ob = pl.multiple_of(row * COLS, 8)
        pltpu.sync_copy(nbuf, o_hbm.at[pl.ds(ob, COLS)])


@functools.partial(jax.jit)
def kernel(logits, u):
    mesh = plsc.VectorSubcoreMesh(core_axis_name="c", subcore_axis_name="s")
    f = functools.partial(
        pl.kernel,
        out_type=jax.ShapeDtypeStruct((ROWS * COLS,), jnp.float32),
        mesh=mesh,
        scratch_types=[
            pltpu.VMEM((COLS,), jnp.float32),
            pltpu.VMEM((CHUNK,), jnp.float32),
            pltpu.VMEM((CHUNK,), jnp.float32),
            pltpu.VMEM((CHUNK,), jnp.float32),
            pltpu.VMEM((CHUNK,), jnp.float32),
            pltpu.SemaphoreType.DMA((4,)),
        ],
    )(_sc_body)
    out = f(logits.reshape(ROWS * COLS), u.reshape(ROWS * COLS))
    return out.reshape(ROWS, COLS)


# speedup vs baseline: 1.9166x; 1.0346x over previous
import functools

import jax
import jax.numpy as jnp
from jax import lax
from jax.experimental import pallas as pl
from jax.experimental.pallas import tpu as pltpu
from jax.experimental.pallas import tpu_sc as plsc

ROWS = 128
COLS = 100000
NC = 2
NS = 16
NW = NC * NS
ROWS_PER_W = ROWS // NW
CHUNK = 4000
NCHUNKS = COLS // CHUNK
VECS = CHUNK // 16
UNROLL = 4

EPS = 1e-11
SHIFT = 16.0
SQRT_BITS = 0x3F3504F3
NEG_LN2 = -0.6931471805599453
P3 = -0.2281064108
P2 = 0.3545640653
P1 = -0.5019703550
P0 = 0.9997000880


def _neg_log(v):
    bits = lax.bitcast_convert_type(v, jnp.int32)
    bp = bits - SQRT_BITS
    e = lax.shift_right_arithmetic(bp, 23)
    fb = (bp & 0x007FFFFF) + SQRT_BITS
    f = lax.bitcast_convert_type(fb, jnp.float32)
    ef = e.astype(jnp.float32)
    r = f - 1.0
    p = ((P3 * r + P2) * r + P1) * r + P0
    return ef * NEG_LN2 - r * p


def _sc_body(x_hbm, u_hbm, o_hbm, nbuf, xb0, xb1, ub0, ub1, sems):
    wid = lax.axis_index("s") * NC + lax.axis_index("c")

    def start_in(ci, row, xb, ub, slot):
        c0 = pl.multiple_of(row * COLS + ci * CHUNK, 8)
        pltpu.make_async_copy(x_hbm.at[pl.ds(c0, CHUNK)], xb,
                              sems.at[slot]).start()
        pltpu.make_async_copy(u_hbm.at[pl.ds(c0, CHUNK)], ub,
                              sems.at[slot + 1]).start()

    def wait_in(ci, row, xb, ub, slot):
        c0 = pl.multiple_of(row * COLS + ci * CHUNK, 8)
        pltpu.make_async_copy(x_hbm.at[pl.ds(c0, CHUNK)], xb,
                              sems.at[slot]).wait()
        pltpu.make_async_copy(u_hbm.at[pl.ds(c0, CHUNK)], ub,
                              sems.at[slot + 1]).wait()

    def chunk_compute(ci, xb, ub, accs):
        base = ci * CHUNK

        @plsc.parallel_loop(0, CHUNK, step=16, unroll=10)
        def pass1(i):
            t = _neg_log(ub[pl.ds(i, 16)])
            n = jnp.exp(xb[pl.ds(i, 16)] - SHIFT) / t
            nbuf[pl.ds(base + i, 16)] = n

        def body(v, accs):
            a0, a1 = accs
            i = base + v * 32
            return (a0 + nbuf[pl.ds(i, 16)],
                    a1 + nbuf[pl.ds(i + 16, 16)])

        return lax.fori_loop(0, CHUNK // 32, body, accs)

    for ri in range(ROWS_PER_W):
        row = wid * ROWS_PER_W + ri
        start_in(0, row, xb0, ub0, 0)

        def pair(k, accs):
            start_in(2 * k + 1, row, xb1, ub1, 2)
            wait_in(2 * k, row, xb0, ub0, 0)
            accs = chunk_compute(2 * k, xb0, ub0, accs)
            start_in(2 * k + 2, row, xb0, ub0, 0)
            wait_in(2 * k + 1, row, xb1, ub1, 2)
            return chunk_compute(2 * k + 1, xb1, ub1, accs)

        zero = jnp.zeros((16,), jnp.float32)
        accs = lax.fori_loop(0, (NCHUNKS - 1) // 2, pair, (zero, zero))
        wait_in(NCHUNKS - 1, row, xb0, ub0, 0)
        accs = chunk_compute(NCHUNKS - 1, xb0, ub0, accs)
        acc = accs[0] + accs[1]

        for _k in (1, 2, 4, 8):
            _idx = jax.lax.iota(jnp.int32, 16) ^ _k
            _perm = jax.lax.gather(
                acc, _idx[:, None],
                jax.lax.GatherDimensionNumbers(
                    offset_dims=(), collapsed_slice_dims=(0,),
                    start_index_map=(0,)),
                (1,), mode=jax.lax.GatherScatterMode.PROMISE_IN_BOUNDS)
            acc = acc + _perm
        inv = 1.0 / acc

        @plsc.parallel_loop(0, COLS, step=16, unroll=5)
        def scale(i):
            sl = pl.ds(i, 16)
            nbuf[sl] = nbuf[sl] * inv
        ob = pl.multiple_of(row * COLS, 8)
        pltpu.sync_copy(nbuf, o_hbm.at[pl.ds(ob, COLS)])


@functools.partial(jax.jit)
def kernel(logits, u):
    mesh = plsc.VectorSubcoreMesh(core_axis_name="c", subcore_axis_name="s")
    f = functools.partial(
        pl.kernel,
        out_type=jax.ShapeDtypeStruct((ROWS * COLS,), jnp.float32),
        mesh=mesh,
        scratch_types=[
            pltpu.VMEM((COLS,), jnp.float32),
            pltpu.VMEM((CHUNK,), jnp.float32),
            pltpu.VMEM((CHUNK,), jnp.float32),
            pltpu.VMEM((CHUNK,), jnp.float32),
            pltpu.VMEM((CHUNK,), jnp.float32),
            pltpu.SemaphoreType.DMA((5,)),
        ],
    )(_sc_body)
    out = f(logits.reshape(ROWS * COLS), u.reshape(ROWS * COLS))
    return out.reshape(ROWS, COLS)
